# Initial kernel scaffold; baseline (speedup 1.0000x reference)
import jax
import jax.numpy as jnp
from jax.experimental import pallas as pl


def kernel(hidden_states, gate_up_weight_packed, gate_up_scales, down_weight_packed, down_scales, expert_ids, expert_probs):
    raise NotImplementedError("write your pallas kernel here")


# speedup vs baseline: 2.2950x; 2.2950x over previous
import jax
import jax.numpy as jnp
from jax import lax
from jax.experimental import pallas as pl

_E = 16
_K = 2048
_I = 1024
_GS = 32
_T = 64
_TOPK = 2
_TI = 256
_NT = _I // _TI


def _dequant(p, s):
    kp, n = p.shape
    g = s.shape[0]
    shifts = lax.broadcasted_iota(jnp.int32, (kp, 8, n), 1) * 4
    nib = lax.shift_right_logical(p[:, None, :], shifts) & 15
    m = nib & 1
    ex = (nib >> 1) & 3
    sg = nib >> 3
    norm = ((ex + 126) << 23) | (m << 22)
    sub = m * 0x3F000000
    bits = (sg << 31) | jnp.where(ex == 0, sub, norm)
    w = lax.bitcast_convert_type(bits, jnp.float32).reshape(kp * 8, n)
    rep = (kp * 8) // g
    se = jnp.broadcast_to(s[:, None, :], (g, rep, n)).reshape(kp * 8, n)
    return w * se


def _moe_kernel(ids_ref, probs_ref, x_ref, gp_ref, gs_ref, up_ref, us_ref,
                dp_ref, ds_ref, o_ref):
    e = pl.program_id(0)
    t = pl.program_id(1)

    @pl.when((e == 0) & (t == 0))
    def _init():
        o_ref[...] = jnp.zeros_like(o_ref)

    x = x_ref[...]
    wg = _dequant(gp_ref[0], gs_ref[0])
    wu = _dequant(up_ref[0], us_ref[0])
    hg = jnp.dot(x, wg, preferred_element_type=jnp.float32)
    hu = jnp.dot(x, wu, preferred_element_type=jnp.float32)
    act = hg * jax.nn.sigmoid(hg) * hu
    wd = _dequant(dp_ref[0], ds_ref[0])
    y = jnp.dot(act, wd, preferred_element_type=jnp.float32)

    w_tok = jnp.sum(jnp.where(ids_ref[...] == e, probs_ref[...], 0.0), axis=1)
    o_ref[...] += y * w_tok[:, None]


def kernel(hidden_states, gate_up_weight_packed, gate_up_scales,
           down_weight_packed, down_scales, expert_ids, expert_probs):
    grid = (_E, _NT)
    out = pl.pallas_call(
        _moe_kernel,
        grid=grid,
        in_specs=[
            pl.BlockSpec((_T, _TOPK), lambda e, t: (0, 0)),
            pl.BlockSpec((_T, _TOPK), lambda e, t: (0, 0)),
            pl.BlockSpec((_T, _K), lambda e, t: (0, 0)),
            pl.BlockSpec((1, _K // 8, _TI), lambda e, t: (e, 0, t)),
            pl.BlockSpec((1, _K // _GS, _TI), lambda e, t: (e, 0, t)),
            pl.BlockSpec((1, _K // 8, _TI), lambda e, t: (e, 0, t + _NT)),
            pl.BlockSpec((1, _K // _GS, _TI), lambda e, t: (e, 0, t + _NT)),
            pl.BlockSpec((1, _TI // 8, _K), lambda e, t: (e, t, 0)),
            pl.BlockSpec((1, _TI // _GS, _K), lambda e, t: (e, t, 0)),
        ],
        out_specs=pl.BlockSpec((_T, _K), lambda e, t: (0, 0)),
        out_shape=jax.ShapeDtypeStruct((_T, _K), jnp.float32),
    )(expert_ids, expert_probs, hidden_states,
      gate_up_weight_packed, gate_up_scales,
      gate_up_weight_packed, gate_up_scales,
      down_weight_packed, down_scales)
    return out


# speedup vs baseline: 3.2928x; 1.4348x over previous
import jax
import jax.numpy as jnp
from jax import lax
from jax.experimental import pallas as pl

_E = 16
_K = 2048
_I = 1024
_GS = 32
_T = 64
_TOPK = 2
_TI = 256
_NT = _I // _TI
_KP = _K // 8
_SUB = 0x3F000000


def _dec(t):
    val3 = t & 7
    mag = jnp.where(val3 < 2, val3 * _SUB, (val3 + 252) << 22)
    return lax.bitcast_convert_type(mag | ((t & 8) << 28), jnp.float32)


def _expand(s, rows, n):
    g = s.shape[0]
    return jnp.broadcast_to(s[:, None, :], (g, rows // g, n)).reshape(rows, n)


def _moe_kernel(ids_ref, probs_ref, x_ref, gp_ref, gs_ref, up_ref, us_ref,
                dp_ref, ds_ref, o_ref):
    e = pl.program_id(0)
    t = pl.program_id(1)

    @pl.when((e == 0) & (t == 0))
    def _init():
        o_ref[...] = jnp.zeros_like(o_ref)

    xb = x_ref[...].astype(jnp.bfloat16)
    gp = gp_ref[0]
    up = up_ref[0]
    se_g = _expand(gs_ref[0], _KP, _TI)
    se_u = _expand(us_ref[0], _KP, _TI)

    hg = jnp.zeros((_T, _TI), jnp.float32)
    hu = jnp.zeros((_T, _TI), jnp.float32)
    for j in range(8):
        gpj = lax.shift_right_logical(gp, 4 * j) if j else gp
        upj = lax.shift_right_logical(up, 4 * j) if j else up
        wg = (_dec(gpj) * se_g).astype(jnp.bfloat16)
        wu = (_dec(upj) * se_u).astype(jnp.bfloat16)
        xj = xb[:, j * _KP:(j + 1) * _KP]
        hg = hg + jnp.dot(xj, wg, preferred_element_type=jnp.float32)
        hu = hu + jnp.dot(xj, wu, preferred_element_type=jnp.float32)
    act = (hg * jax.nn.sigmoid(hg) * hu).astype(jnp.bfloat16)

    dp = dp_ref[0]
    kp, n = dp.shape
    shifts = lax.broadcasted_iota(jnp.int32, (kp, 8, n), 1) * 4
    nib = lax.shift_right_logical(dp[:, None, :], shifts)
    wd = (_dec(nib).reshape(kp * 8, n) * _expand(ds_ref[0], _TI, _K)
          ).astype(jnp.bfloat16)
    y = jnp.dot(act, wd, preferred_element_type=jnp.float32)

    w_tok = jnp.sum(jnp.where(ids_ref[...] == e, probs_ref[...], 0.0), axis=1)
    o_ref[...] += y * w_tok[:, None]


def kernel(hidden_states, gate_up_weight_packed, gate_up_scales,
           down_weight_packed, down_scales, expert_ids, expert_probs):
    xp = hidden_states.reshape(_T, _KP, 8).transpose(0, 2, 1).reshape(_T, _K)
    grid = (_E, _NT)
    out = pl.pallas_call(
        _moe_kernel,
        grid=grid,
        in_specs=[
            pl.BlockSpec((_T, _TOPK), lambda e, t: (0, 0)),
            pl.BlockSpec((_T, _TOPK), lambda e, t: (0, 0)),
            pl.BlockSpec((_T, _K), lambda e, t: (0, 0)),
            pl.BlockSpec((1, _KP, _TI), lambda e, t: (e, 0, t)),
            pl.BlockSpec((1, _K // _GS, _TI), lambda e, t: (e, 0, t)),
            pl.BlockSpec((1, _KP, _TI), lambda e, t: (e, 0, t + _NT)),
            pl.BlockSpec((1, _K // _GS, _TI), lambda e, t: (e, 0, t + _NT)),
            pl.BlockSpec((1, _TI // 8, _K), lambda e, t: (e, t, 0)),
            pl.BlockSpec((1, _TI // _GS, _K), lambda e, t: (e, t, 0)),
        ],
        out_specs=pl.BlockSpec((_T, _K), lambda e, t: (0, 0)),
        out_shape=jax.ShapeDtypeStruct((_T, _K), jnp.float32),
    )(expert_ids, expert_probs, xp,
      gate_up_weight_packed, gate_up_scales,
      gate_up_weight_packed, gate_up_scales,
      down_weight_packed, down_scales)
    return out


# speedup vs baseline: 3.3026x; 1.0030x over previous
import jax
import jax.numpy as jnp
from jax import lax
from jax.experimental import pallas as pl

_E = 16
_K = 2048
_I = 1024
_GS = 32
_T = 64
_TOPK = 2
_TI = 1024
_NT = _I // _TI
_KP = _K // 8
_SUB = 0x3F000000


def _dec(t):
    val3 = t & 7
    mag = jnp.where(val3 < 2, val3 * _SUB, (val3 + 252) << 22)
    return lax.bitcast_convert_type(mag | ((t & 8) << 28), jnp.float32)


def _expand(s, rows, n):
    g = s.shape[0]
    return jnp.broadcast_to(s[:, None, :], (g, rows // g, n)).reshape(rows, n)


def _moe_kernel(ids_ref, probs_ref, x_ref, gp_ref, gs_ref, up_ref, us_ref,
                dp_ref, ds_ref, o_ref):
    e = pl.program_id(0)
    t = pl.program_id(1)

    @pl.when((e == 0) & (t == 0))
    def _init():
        o_ref[...] = jnp.zeros_like(o_ref)

    xb = x_ref[...].astype(jnp.bfloat16)
    gp = gp_ref[0]
    up = up_ref[0]
    se_g = _expand(gs_ref[0], _KP, _TI)
    se_u = _expand(us_ref[0], _KP, _TI)

    hg = jnp.zeros((_T, _TI), jnp.float32)
    hu = jnp.zeros((_T, _TI), jnp.float32)
    for j in range(8):
        gpj = lax.shift_right_logical(gp, 4 * j) if j else gp
        upj = lax.shift_right_logical(up, 4 * j) if j else up
        wg = (_dec(gpj) * se_g).astype(jnp.bfloat16)
        wu = (_dec(upj) * se_u).astype(jnp.bfloat16)
        xj = xb[:, j * _KP:(j + 1) * _KP]
        hg = hg + jnp.dot(xj, wg, preferred_element_type=jnp.float32)
        hu = hu + jnp.dot(xj, wu, preferred_element_type=jnp.float32)
    act = (hg * jax.nn.sigmoid(hg) * hu).astype(jnp.bfloat16)

    dp = dp_ref[0]
    kp, n = dp.shape
    shifts = lax.broadcasted_iota(jnp.int32, (kp, 8, n), 1) * 4
    nib = lax.shift_right_logical(dp[:, None, :], shifts)
    wd = (_dec(nib).reshape(kp * 8, n) * _expand(ds_ref[0], _TI, _K)
          ).astype(jnp.bfloat16)
    y = jnp.dot(act, wd, preferred_element_type=jnp.float32)

    w_tok = jnp.sum(jnp.where(ids_ref[...] == e, probs_ref[...], 0.0), axis=1)
    o_ref[...] += y * w_tok[:, None]


def kernel(hidden_states, gate_up_weight_packed, gate_up_scales,
           down_weight_packed, down_scales, expert_ids, expert_probs):
    xp = hidden_states.reshape(_T, _KP, 8).transpose(0, 2, 1).reshape(_T, _K)
    grid = (_E, _NT)
    out = pl.pallas_call(
        _moe_kernel,
        grid=grid,
        in_specs=[
            pl.BlockSpec((_T, _TOPK), lambda e, t: (0, 0)),
            pl.BlockSpec((_T, _TOPK), lambda e, t: (0, 0)),
            pl.BlockSpec((_T, _K), lambda e, t: (0, 0)),
            pl.BlockSpec((1, _KP, _TI), lambda e, t: (e, 0, t)),
            pl.BlockSpec((1, _K // _GS, _TI), lambda e, t: (e, 0, t)),
            pl.BlockSpec((1, _KP, _TI), lambda e, t: (e, 0, t + _NT)),
            pl.BlockSpec((1, _K // _GS, _TI), lambda e, t: (e, 0, t + _NT)),
            pl.BlockSpec((1, _TI // 8, _K), lambda e, t: (e, t, 0)),
            pl.BlockSpec((1, _TI // _GS, _K), lambda e, t: (e, t, 0)),
        ],
        out_specs=pl.BlockSpec((_T, _K), lambda e, t: (0, 0)),
        out_shape=jax.ShapeDtypeStruct((_T, _K), jnp.float32),
    )(expert_ids, expert_probs, xp,
      gate_up_weight_packed, gate_up_scales,
      gate_up_weight_packed, gate_up_scales,
      down_weight_packed, down_scales)
    return out
